# baseline (speedup 1.0000x reference)
import functools

import jax
import jax.numpy as jnp
from jax import lax
from jax.experimental import pallas as pl
from jax.experimental.pallas import tpu as pltpu
from jax.experimental.pallas import tpu_sc as plsc

DIM = 64
LANES = 16
IDXW = 128
CHUNK = 256
GPC = CHUNK // IDXW


@functools.lru_cache(maxsize=None)
def _build(n_rows: int):
    info = plsc.get_sparse_core_info()
    num_workers = info.num_cores * info.num_subcores
    per_w = n_rows // num_workers
    n_chunks = per_w // CHUNK
    assert per_w * num_workers == n_rows and n_chunks * CHUNK == per_w
    assert n_chunks % 2 == 0
    half = n_chunks // 2

    mesh = plsc.VectorSubcoreMesh(core_axis_name="c", subcore_axis_name="s")

    @functools.partial(
        pl.kernel,
        mesh=mesh,
        compiler_params=pltpu.CompilerParams(use_tc_tiling_on_sc=False),
        out_type=jax.ShapeDtypeStruct((n_rows, DIM), jnp.float32),
        scratch_types=[
            [pltpu.VMEM((GPC, IDXW), jnp.int32) for _ in range(2)],
            [pltpu.VMEM((GPC, IDXW), jnp.int32) for _ in range(2)],
            [pltpu.VMEM((GPC, IDXW), jnp.int32) for _ in range(2)],
            [pltpu.VMEM((CHUNK, DIM), jnp.float32) for _ in range(2)],
            [pltpu.VMEM((CHUNK, DIM), jnp.float32) for _ in range(2)],
            [pltpu.VMEM((CHUNK, DIM), jnp.float32) for _ in range(2)],
            [pltpu.SemaphoreType.DMA for _ in range(2)],
            [pltpu.SemaphoreType.DMA for _ in range(2)],
            [pltpu.SemaphoreType.DMA for _ in range(2)],
        ],
    )
    def emb_sum(gid, cid, rid, basic, homo, rna, out,
                gi_v, ci_v, ri_v, acc_v, h_v, r_v,
                sem_idx, sem_g, sem_st):
        w = lax.axis_index("s") * info.num_cores + lax.axis_index("c")
        wrow = w * n_chunks * GPC
        wbase = w * per_w

        def fire_idx(g, p):
            pltpu.async_copy(gid.at[pl.ds(wrow + g * GPC, GPC)], gi_v[p],
                             sem_idx[p])
            pltpu.async_copy(cid.at[pl.ds(wrow + g * GPC, GPC)], ci_v[p],
                             sem_idx[p])
            pltpu.async_copy(rid.at[pl.ds(wrow + g * GPC, GPC)], ri_v[p],
                             sem_idx[p])

        def wait_idx(p):
            for _ in range(3):
                pltpu.make_async_copy(gid.at[pl.ds(0, GPC)], gi_v[p],
                                      sem_idx[p]).wait()

        def fire_gathers(p):
            for j in range(GPC):
                dst = pl.ds(j * IDXW, IDXW)
                pltpu.async_copy(basic.at[gi_v[p].at[j]], acc_v[p].at[dst],
                                 sem_g[p])
                pltpu.async_copy(homo.at[ci_v[p].at[j]], h_v[p].at[dst],
                                 sem_g[p])
                pltpu.async_copy(rna.at[ri_v[p].at[j]], r_v[p].at[dst],
                                 sem_g[p])

        def wait_gathers(p):
            for _ in range(3 * GPC):
                pltpu.make_async_copy(basic.at[gi_v[p].at[0]],
                                      acc_v[p].at[pl.ds(0, IDXW)],
                                      sem_g[p]).wait()

        def fire_store(g, p):
            pltpu.async_copy(acc_v[p],
                             out.at[pl.ds(wbase + g * CHUNK, CHUNK)],
                             sem_st[p])

        def wait_store(p):
            pltpu.make_async_copy(acc_v[p], out.at[pl.ds(0, CHUNK)],
                                  sem_st[p]).wait()

        def compute(p):
            acc, h, r = acc_v[p], h_v[p], r_v[p]

            @plsc.parallel_loop(0, CHUNK, step=1, unroll=8)
            def row_body(rr):
                for c in range(DIM // LANES):
                    sl = pl.ds(c * LANES, LANES)
                    plsc.addupdate(acc.at[rr, sl], h[rr, sl] + r[rr, sl])

        fire_idx(0, 0)
        fire_idx(1, 1)
        wait_idx(0)
        fire_gathers(0)

        def body(t, carry):
            a = 2 * t

            wait_idx(1)

            @pl.when(t > 0)
            def _():
                wait_store(1)

            fire_gathers(1)
            wait_gathers(0)

            @pl.when(t < half - 1)
            def _():
                fire_idx(a + 2, 0)

            compute(0)
            fire_store(a, 0)

            @pl.when(t < half - 1)
            def _():
                wait_idx(0)

            wait_store(0)

            @pl.when(t < half - 1)
            def _():
                fire_gathers(0)

            wait_gathers(1)
            compute(1)
            fire_store(a + 1, 1)

            @pl.when(t < half - 1)
            def _():
                fire_idx(a + 3, 1)

            return carry

        lax.fori_loop(0, half, body, 0)
        wait_store(1)

    return emb_sum


def kernel(x_gene_id, x_connect_id, x_rna_type, basic_table, homo_table, rna_table):
    batch, seq = x_gene_id.shape
    n = batch * seq
    gid = x_gene_id.reshape(n // IDXW, IDXW).astype(jnp.int32)
    cid = x_connect_id.reshape(n // IDXW, IDXW).astype(jnp.int32)
    rid = x_rna_type.reshape(n // IDXW, IDXW).astype(jnp.int32)
    out = _build(n)(gid, cid, rid, basic_table, homo_table, rna_table)
    return out.reshape(batch, seq, DIM)
